# Initial kernel scaffold; baseline (speedup 1.0000x reference)
import jax
import jax.numpy as jnp
from jax.experimental import pallas as pl


def kernel(actions, emb_table, W, b):
    raise NotImplementedError("write your pallas kernel here")


# speedup vs baseline: 2.7201x; 2.7201x over previous
import functools

import jax
import jax.numpy as jnp
from jax import lax
from jax.experimental import pallas as pl
from jax.experimental.pallas import tpu as pltpu
from jax.experimental.pallas import tpu_sc as plsc

BATCH = 16384
HIST = 50
D = 64

NC = 2
NS = 16
NW = NC * NS

ROWS_PER_W = BATCH // NW
CHUNK_ROWS = 8
IDX_PER_CHUNK = CHUNK_ROWS * HIST
NCHUNK = ROWS_PER_W // CHUNK_ROWS
LANES = 16
DSUB = D // LANES


def _sc_body(actions_hbm, table_hbm, out_hbm,
             idx_v, buf0, buf1, out_v, sem0, sem1):
    wid = lax.axis_index("s") * NC + lax.axis_index("c")
    base_row = wid * ROWS_PER_W
    base_idx = base_row * HIST

    pltpu.sync_copy(actions_hbm.at[pl.ds(pl.multiple_of(base_idx, 8),
                                         ROWS_PER_W * HIST)], idx_v)

    bufs = (buf0, buf1)
    sems = (sem0, sem1)

    def _start_gather(chunk, buf, sem):
        off = pl.multiple_of(chunk * IDX_PER_CHUNK, 8)
        return pltpu.async_copy(
            table_hbm.at[idx_v.at[pl.ds(off, IDX_PER_CHUNK)]], buf, sem)

    _start_gather(0, buf0, sem0)
    _start_gather(1, buf1, sem1)

    def _reduce_chunk(chunk, buf):
        def _row(r, _):
            row0 = r * HIST
            acc = [buf[row0, pl.ds(j * LANES, LANES)] for j in range(DSUB)]
            for i in range(1, HIST):
                for j in range(DSUB):
                    acc[j] = acc[j] + buf[row0 + i, pl.ds(j * LANES, LANES)]
            orow = chunk * CHUNK_ROWS + r
            for j in range(DSUB):
                out_v[orow, pl.ds(j * LANES, LANES)] = acc[j]
            return _
        lax.fori_loop(0, CHUNK_ROWS, _row, 0, unroll=False)

    def _step(i, carry):
        for p in range(2):
            chunk = 2 * i + p
            pltpu.make_async_copy(
                table_hbm.at[idx_v.at[pl.ds(0, IDX_PER_CHUNK)]],
                bufs[p], sems[p]).wait()
            _reduce_chunk(chunk, bufs[p])

            @pl.when(i < NCHUNK // 2 - 1)
            def _start_next(p=p, chunk=chunk):
                _start_gather(chunk + 2, bufs[p], sems[p])
        return carry

    lax.fori_loop(0, NCHUNK // 2, _step, 0, unroll=False)

    pltpu.sync_copy(out_v,
                    out_hbm.at[pl.ds(pl.multiple_of(base_row, 8),
                                     ROWS_PER_W)])


@functools.partial(jax.jit, static_argnums=())
def _sc_gather_pool(actions_flat, table):
    mesh = plsc.VectorSubcoreMesh(core_axis_name="c", subcore_axis_name="s",
                                  num_cores=NC, num_subcores=NS)
    fn = pl.kernel(
        _sc_body,
        out_type=jax.ShapeDtypeStruct((BATCH, D), jnp.float32),
        mesh=mesh,
        compiler_params=pltpu.CompilerParams(use_tc_tiling_on_sc=False),
        scratch_types=[
            pltpu.VMEM((ROWS_PER_W * HIST,), jnp.int32),
            pltpu.VMEM((IDX_PER_CHUNK, D), jnp.float32),
            pltpu.VMEM((IDX_PER_CHUNK, D), jnp.float32),
            pltpu.VMEM((ROWS_PER_W, D), jnp.float32),
            pltpu.SemaphoreType.DMA,
            pltpu.SemaphoreType.DMA,
        ],
    )
    return fn(actions_flat, table)


def _tc_project_body(x_ref, w_ref, b_ref, o_ref):
    x = x_ref[...] * (1.0 / HIST)
    o_ref[...] = lax.dot_general(
        x, w_ref[...], (((1,), (1,)), ((), ())),
        preferred_element_type=jnp.float32) + b_ref[...]


def _tc_project(pooled, w, b2):
    bm = 1024
    return pl.pallas_call(
        _tc_project_body,
        grid=(BATCH // bm,),
        in_specs=[
            pl.BlockSpec((bm, D), lambda i: (i, 0)),
            pl.BlockSpec((D, D), lambda i: (0, 0)),
            pl.BlockSpec((1, D), lambda i: (0, 0)),
        ],
        out_specs=pl.BlockSpec((bm, D), lambda i: (i, 0)),
        out_shape=jax.ShapeDtypeStruct((BATCH, D), jnp.float32),
    )(pooled, w, b2)


def kernel(actions, emb_table, W, b):
    actions_flat = actions.reshape(-1).astype(jnp.int32)
    pooled = _sc_gather_pool(actions_flat, emb_table)
    return _tc_project(pooled, W, b.reshape(1, D))
